# baseline (speedup 1.0000x reference)
import functools

import jax
import jax.numpy as jnp
from jax import lax
from jax.experimental import pallas as pl
from jax.experimental.pallas import tpu as pltpu
from jax.experimental.pallas import tpu_sc as plsc

N = 10000
E = 320000
D = 128
H = 128
DE = 16

BN = 2000
BE = 2560

_NC = 2
_NS = 16
_NW = _NC * _NS
_CG = 80
_EPW = E // _NW
_NCH = _EPW // _CG
_L = 16


def _silu(x):
    return x / (1.0 + jnp.exp(-x))


def _ln(x, g, b, eps=1e-5):
    mu = jnp.mean(x, axis=-1, keepdims=True)
    var = jnp.mean((x - mu) ** 2, axis=-1, keepdims=True)
    return (x - mu) * jax.lax.rsqrt(var + eps) * g + b


def _k1_body(h_ref, g_ref, b_ref, w1r_ref, w1c_ref,
             hn_ref, pr_ref, pc_ref):
    h = h_ref[:]
    hn = _ln(h, g_ref[:], b_ref[:])
    hn_ref[:] = hn
    pr_ref[:] = jnp.dot(hn, w1r_ref[:],
                        preferred_element_type=jnp.float32).astype(jnp.bfloat16)
    pc_ref[:] = jnp.dot(hn, w1c_ref[:],
                        preferred_element_type=jnp.float32).astype(jnp.bfloat16)


def _k1(h, g, b, w1r, w1c):
    grid = N // BN
    return pl.pallas_call(
        _k1_body,
        grid=(grid,),
        in_specs=[
            pl.BlockSpec((BN, D), lambda i: (i, 0)),
            pl.BlockSpec((D,), lambda i: (0,)),
            pl.BlockSpec((D,), lambda i: (0,)),
            pl.BlockSpec((D, H), lambda i: (0, 0)),
            pl.BlockSpec((D, H), lambda i: (0, 0)),
        ],
        out_specs=[
            pl.BlockSpec((BN, D), lambda i: (i, 0)),
            pl.BlockSpec((BN, H), lambda i: (i, 0)),
            pl.BlockSpec((BN, H), lambda i: (i, 0)),
        ],
        out_shape=[
            jax.ShapeDtypeStruct((N, D), jnp.float32),
            jax.ShapeDtypeStruct((N, H), jnp.bfloat16),
            jax.ShapeDtypeStruct((N, H), jnp.bfloat16),
        ],
    )(h, g, b, w1r, w1c)


def _k2_body(pr, pc, cx, cy, cz, row, col, z1, gr, gc, cdt, cntp,
             idxr, idxc, grb, gcb, crx, cry, crz, ccx, ccy, ccz,
             planes, ones, cnt_sp, sem):
    c = lax.axis_index("c")
    s = lax.axis_index("s")
    wid = s * _NC + c
    base = wid * _EPW
    pltpu.sync_copy(z1, cnt_sp.at[pl.ds(s * _TSL, _TSL)])
    for j in range(_CG // _L):
        ones[pl.ds(j * _L, _L)] = jnp.full((_L,), 1.0, jnp.float32)
    plsc.subcore_barrier()

    def chunk(i, carry):
        off = i * _CG
        pltpu.sync_copy(row.at[pl.ds(base + off, _CG)], idxr)
        pltpu.sync_copy(col.at[pl.ds(base + off, _CG)], idxc)
        cps = [
            pltpu.async_copy(pr.at[idxr], grb, sem),
            pltpu.async_copy(pc.at[idxc], gcb, sem),
            pltpu.async_copy(cx.at[idxr], crx, sem),
            pltpu.async_copy(cy.at[idxr], cry, sem),
            pltpu.async_copy(cz.at[idxr], crz, sem),
            pltpu.async_copy(cx.at[idxc], ccx, sem),
            pltpu.async_copy(cy.at[idxc], ccy, sem),
            pltpu.async_copy(cz.at[idxc], ccz, sem),
        ]
        for cp in cps:
            cp.wait()
        for j in range(_CG // _L):
            sj = pl.ds(j * _L, _L)
            so = pl.ds(off + j * _L, _L)
            dx = crx[sj] - ccx[sj]
            dy = cry[sj] - ccy[sj]
            dz = crz[sj] - ccz[sj]
            planes[0, so] = dx
            planes[1, so] = dy
            planes[2, so] = dz
            planes[3, so] = dx * dx + dy * dy + dz * dz
        pltpu.sync_copy(grb, gr.at[pl.ds(base + off, _CG)])
        pltpu.sync_copy(gcb, gc.at[pl.ds(base + off, _CG)])
        pltpu.sync_copy(ones, cnt_sp.at[idxr], add=True)
        return carry

    lax.fori_loop(0, _NCH, chunk, 0)
    for d in range(4):
        pltpu.sync_copy(planes.at[d], cdt.at[d, pl.ds(base, _EPW)])
    plsc.subcore_barrier()
    pltpu.sync_copy(cnt_sp.at[pl.ds(s * _TSL, _TSL)],
                    cntp.at[c, pl.ds(s * _TSL, _TSL)])


def _k2(pr, pc, cx, cy, cz, row, col, z1):
    mesh = plsc.VectorSubcoreMesh(core_axis_name="c", subcore_axis_name="s")
    f = pl.kernel(
        _k2_body,
        out_type=[
            jax.ShapeDtypeStruct((E, H), jnp.bfloat16),
            jax.ShapeDtypeStruct((E, H), jnp.bfloat16),
            jax.ShapeDtypeStruct((4, E), jnp.float32),
            jax.ShapeDtypeStruct((2, NPAD), jnp.float32),
        ],
        mesh=mesh,
        scratch_types=[
            pltpu.VMEM((_CG,), jnp.int32),
            pltpu.VMEM((_CG,), jnp.int32),
            pltpu.VMEM((_CG, H), jnp.bfloat16),
            pltpu.VMEM((_CG, H), jnp.bfloat16),
            pltpu.VMEM((_CG,), jnp.float32),
            pltpu.VMEM((_CG,), jnp.float32),
            pltpu.VMEM((_CG,), jnp.float32),
            pltpu.VMEM((_CG,), jnp.float32),
            pltpu.VMEM((_CG,), jnp.float32),
            pltpu.VMEM((_CG,), jnp.float32),
            pltpu.VMEM((4, _EPW), jnp.float32),
            pltpu.VMEM((_CG,), jnp.float32),
            pltpu.VMEM_SHARED((NPAD,), jnp.float32),
            pltpu.SemaphoreType.DMA,
        ],
        compiler_params=pltpu.CompilerParams(use_tc_tiling_on_sc=False),
    )
    return f(pr, pc, cx, cy, cz, row, col, z1)


NPAD = 10240
_TSL = NPAD // _NS
NAGG = 10000
_TSA = NAGG // _NS
_CS = 80


def _k4_body(ef, trt, row, z128, z1, aggp, trp,
             idxb, efb, trpl, agg_sp, t0_sp, t1_sp, t2_sp):
    c = lax.axis_index("c")
    s = lax.axis_index("s")
    wid = s * _NC + c
    base = wid * _EPW
    tsp = [t0_sp, t1_sp, t2_sp]

    pltpu.sync_copy(z128, agg_sp.at[pl.ds(s * _TSA, _TSA)])
    for d in range(3):
        pltpu.sync_copy(z1, tsp[d].at[pl.ds(s * _TSL, _TSL)])
        pltpu.sync_copy(trt.at[d, pl.ds(base, _EPW)], trpl.at[d])
    plsc.subcore_barrier()

    def chunk(i, carry):
        off = base + i * _CS
        pltpu.sync_copy(row.at[pl.ds(off, _CS)], idxb)
        pltpu.sync_copy(ef.at[pl.ds(off, _CS)], efb)
        pltpu.sync_copy(efb, agg_sp.at[idxb], add=True)
        for d in range(3):
            pltpu.sync_copy(trpl.at[d, pl.ds(i * _CS, _CS)],
                            tsp[d].at[idxb], add=True)
        return carry

    lax.fori_loop(0, _EPW // _CS, chunk, 0)
    plsc.subcore_barrier()

    pltpu.sync_copy(agg_sp.at[pl.ds(s * _TSA, _TSA)],
                    aggp.at[c, pl.ds(s * _TSA, _TSA)])
    for d in range(3):
        pltpu.sync_copy(tsp[d].at[pl.ds(s * _TSL, _TSL)],
                        trp.at[c, d, pl.ds(s * _TSL, _TSL)])


def _k4(ef, trt, row, z128, z1):
    mesh = plsc.VectorSubcoreMesh(core_axis_name="c", subcore_axis_name="s")
    f = pl.kernel(
        _k4_body,
        out_type=[
            jax.ShapeDtypeStruct((2, NAGG, H), jnp.float32),
            jax.ShapeDtypeStruct((2, 3, NPAD), jnp.float32),
        ],
        mesh=mesh,
        scratch_types=[
            pltpu.VMEM((_CS,), jnp.int32),
            pltpu.VMEM((_CS, H), jnp.float32),
            pltpu.VMEM((3, _EPW), jnp.float32),
            pltpu.VMEM_SHARED((NAGG, H), jnp.float32),
            pltpu.VMEM_SHARED((NPAD,), jnp.float32),
            pltpu.VMEM_SHARED((NPAD,), jnp.float32),
            pltpu.VMEM_SHARED((NPAD,), jnp.float32),
        ],
        compiler_params=pltpu.CompilerParams(use_tc_tiling_on_sc=False),
    )
    return f(ef, trt, row, z128, z1)


def _k3_body(gr_ref, gc_ref, cd_ref, ea_ref,
             w8_ref, w1e_ref, b1_ref, w2_ref, b2_ref,
             lng_ref, lnb_ref, cw1_ref, cb1_ref, cw2_ref,
             ef_ref, trt_ref):
    cd = cd_ref[:]
    pre1 = (gr_ref[:].astype(jnp.float32) + gc_ref[:].astype(jnp.float32)
            + jax.lax.dot_general(cd, w8_ref[:], (((0,), (0,)), ((), ())),
                                  preferred_element_type=jnp.float32)
            + jax.lax.dot_general(ea_ref[:], w1e_ref[:], (((0,), (0,)), ((), ())),
                                  preferred_element_type=jnp.float32)
            + b1_ref[:])
    x1 = _silu(pre1)
    x2 = _silu(jnp.dot(x1, w2_ref[:], preferred_element_type=jnp.float32) + b2_ref[:])
    ef = _ln(x2, lng_ref[:], lnb_ref[:])
    ef_ref[:] = ef
    s2 = _silu(jnp.dot(ef, cw1_ref[:], preferred_element_type=jnp.float32) + cb1_ref[:])
    cm_row = jax.lax.dot_general(cw2_ref[:], s2, (((0,), (1,)), ((), ())),
                                 preferred_element_type=jnp.float32)
    trt_ref[:] = cd * cm_row


def _k3(gr, gc, cdt, ea_t, w8, w1e, b1, w2, b2, lng, lnb, cw1, cb1, cw2):
    grid = E // BE
    full = lambda shape: pl.BlockSpec(shape, lambda i: tuple(0 for _ in shape))
    return pl.pallas_call(
        _k3_body,
        grid=(grid,),
        in_specs=[
            pl.BlockSpec((BE, H), lambda i: (i, 0)),
            pl.BlockSpec((BE, H), lambda i: (i, 0)),
            pl.BlockSpec((4, BE), lambda i: (0, i)),
            pl.BlockSpec((DE, BE), lambda i: (0, i)),
            full((4, H)),
            full((DE, H)),
            full((H,)),
            full((H, H)),
            full((H,)),
            full((H,)),
            full((H,)),
            full((H, H)),
            full((H,)),
            full((H, 1)),
        ],
        out_specs=[
            pl.BlockSpec((BE, H), lambda i: (i, 0)),
            pl.BlockSpec((4, BE), lambda i: (0, i)),
        ],
        out_shape=[
            jax.ShapeDtypeStruct((E, H), jnp.float32),
            jax.ShapeDtypeStruct((4, E), jnp.float32),
        ],
    )(gr, gc, cdt, ea_t, w8, w1e, b1, w2, b2, lng, lnb, cw1, cb1, cw2)


def _k5_body(h_ref, hn_ref, agg_ref,
             w1h_ref, w1a_ref, b1_ref, w2_ref, b2_ref,
             hout_ref):
    agg = agg_ref[0] + agg_ref[1]
    pre = (jnp.dot(hn_ref[:], w1h_ref[:], preferred_element_type=jnp.float32)
           + jnp.dot(agg, w1a_ref[:], preferred_element_type=jnp.float32)
           + b1_ref[:])
    nh = jnp.dot(_silu(pre), w2_ref[:], preferred_element_type=jnp.float32) + b2_ref[:]
    hout_ref[:] = h_ref[:] + nh


def _k5(h, hn, aggp, w1h, w1a, b1, w2, b2):
    grid = N // BN
    full = lambda shape: pl.BlockSpec(shape, lambda i: tuple(0 for _ in shape))
    return pl.pallas_call(
        _k5_body,
        grid=(grid,),
        in_specs=[
            pl.BlockSpec((BN, D), lambda i: (i, 0)),
            pl.BlockSpec((BN, D), lambda i: (i, 0)),
            pl.BlockSpec((2, BN, H), lambda i: (0, i, 0)),
            full((D, H)),
            full((H, H)),
            full((H,)),
            full((H, D)),
            full((D,)),
        ],
        out_specs=[
            pl.BlockSpec((BN, D), lambda i: (i, 0)),
        ],
        out_shape=[
            jax.ShapeDtypeStruct((N, D), jnp.float32),
        ],
    )(h, hn, aggp, w1h, w1a, b1, w2, b2)


def _k6_body(trp_ref, cnt_ref, ct_ref, cout_ref):
    tr = trp_ref[0] + trp_ref[1]
    cnt = jnp.maximum(cnt_ref[0:1, :N] + cnt_ref[1:2, :N], 1.0)
    cout_ref[:] = ct_ref[:] + tr[0:3, :N] / cnt


def _k6(trp, cntp, coord_t):
    return pl.pallas_call(
        _k6_body,
        out_shape=jax.ShapeDtypeStruct((3, N), jnp.float32),
    )(trp, cntp, coord_t)


def kernel(h, edge_index, coord, edge_attr, node_ln_g, node_ln_b,
           edge_ln_g, edge_ln_b, e_W1, e_b1, e_W2, e_b2,
           n_W1, n_b1, n_W2, n_b2, c_W1, c_b1, c_W2):
    row = edge_index[0]
    col = edge_index[1]
    w1r = e_W1[0:D]
    w1c = e_W1[D:2 * D]
    wrad = e_W1[2 * D]
    w1e = e_W1[2 * D + 1:]
    w8 = jnp.zeros((4, H), jnp.float32).at[3].set(wrad)
    ea_t = edge_attr.T
    coord_t = coord.T

    hn, pr, pc = _k1(h, node_ln_g, node_ln_b, w1r, w1c)

    z1 = jnp.zeros((_TSL,), jnp.float32)
    gr, gc, cdt, cntp = _k2(pr, pc, coord_t[0], coord_t[1], coord_t[2],
                            row, col, z1)

    ef, trt = _k3(gr, gc, cdt, ea_t, w8, w1e, e_b1, e_W2, e_b2,
                  edge_ln_g, edge_ln_b, c_W1, c_b1, c_W2)

    z128 = jnp.zeros((_TSA, H), jnp.float32)
    aggp, trp = _k4(ef, trt, row, z128, z1)

    (h_out,) = _k5(h, hn, aggp, n_W1[0:D], n_W1[D:], n_b1, n_W2, n_b2)
    coord_out_t = _k6(trp, cntp, coord_t)
    return (h_out, coord_out_t.T, edge_attr)


# speedup vs baseline: 1.5318x; 1.5318x over previous
import functools

import jax
import jax.numpy as jnp
from jax import lax
from jax.experimental import pallas as pl
from jax.experimental.pallas import tpu as pltpu
from jax.experimental.pallas import tpu_sc as plsc

N = 10000
E = 320000
D = 128
H = 128
DE = 16

BN = 2000
BE = 2560

_NC = 2
_NS = 16
_NW = _NC * _NS
_CG = 80
_EPW = E // _NW
_NCH = _EPW // _CG
_L = 16


def _silu(x):
    return x / (1.0 + jnp.exp(-x))


def _ln(x, g, b, eps=1e-5):
    mu = jnp.mean(x, axis=-1, keepdims=True)
    var = jnp.mean((x - mu) ** 2, axis=-1, keepdims=True)
    return (x - mu) * jax.lax.rsqrt(var + eps) * g + b


def _k1_body(h_ref, g_ref, b_ref, w1r_ref, w1c_ref,
             hn_ref, pr_ref, pc_ref):
    h = h_ref[:]
    hn = _ln(h, g_ref[:], b_ref[:])
    hn_ref[:] = hn
    pr_ref[:] = jnp.dot(hn, w1r_ref[:], preferred_element_type=jnp.float32)
    pc_ref[:] = jnp.dot(hn, w1c_ref[:], preferred_element_type=jnp.float32)


def _k1(h, g, b, w1r, w1c):
    grid = N // BN
    return pl.pallas_call(
        _k1_body,
        grid=(grid,),
        in_specs=[
            pl.BlockSpec((BN, D), lambda i: (i, 0)),
            pl.BlockSpec((D,), lambda i: (0,)),
            pl.BlockSpec((D,), lambda i: (0,)),
            pl.BlockSpec((D, H), lambda i: (0, 0)),
            pl.BlockSpec((D, H), lambda i: (0, 0)),
        ],
        out_specs=[
            pl.BlockSpec((BN, D), lambda i: (i, 0)),
            pl.BlockSpec((BN, H), lambda i: (i, 0)),
            pl.BlockSpec((BN, H), lambda i: (i, 0)),
        ],
        out_shape=[
            jax.ShapeDtypeStruct((N, D), jnp.float32),
            jax.ShapeDtypeStruct((N, H), jnp.float32),
            jax.ShapeDtypeStruct((N, H), jnp.float32),
        ],
    )(h, g, b, w1r, w1c)


def _k2_body(pr, pc, cx, cy, cz, row, col, z1, gr, gc, cdt, cntp,
             idxr, idxc, grb, gcb, crx, cry, crz, ccx, ccy, ccz,
             planes, ones, cnt_sp, sem):
    c = lax.axis_index("c")
    s = lax.axis_index("s")
    wid = s * _NC + c
    base = wid * _EPW
    pltpu.sync_copy(z1, cnt_sp.at[pl.ds(s * _TSL, _TSL)])
    for j in range(_CG // _L):
        ones[pl.ds(j * _L, _L)] = jnp.full((_L,), 1.0, jnp.float32)
    plsc.subcore_barrier()

    def chunk(i, carry):
        off = i * _CG
        pltpu.sync_copy(row.at[pl.ds(base + off, _CG)], idxr)
        pltpu.sync_copy(col.at[pl.ds(base + off, _CG)], idxc)
        cps = [
            pltpu.async_copy(pr.at[idxr], grb, sem),
            pltpu.async_copy(pc.at[idxc], gcb, sem),
            pltpu.async_copy(cx.at[idxr], crx, sem),
            pltpu.async_copy(cy.at[idxr], cry, sem),
            pltpu.async_copy(cz.at[idxr], crz, sem),
            pltpu.async_copy(cx.at[idxc], ccx, sem),
            pltpu.async_copy(cy.at[idxc], ccy, sem),
            pltpu.async_copy(cz.at[idxc], ccz, sem),
        ]
        for cp in cps:
            cp.wait()
        for j in range(_CG // _L):
            sj = pl.ds(j * _L, _L)
            so = pl.ds(off + j * _L, _L)
            dx = crx[sj] - ccx[sj]
            dy = cry[sj] - ccy[sj]
            dz = crz[sj] - ccz[sj]
            planes[0, so] = dx
            planes[1, so] = dy
            planes[2, so] = dz
            planes[3, so] = dx * dx + dy * dy + dz * dz
        pltpu.sync_copy(grb, gr.at[pl.ds(base + off, _CG)])
        pltpu.sync_copy(gcb, gc.at[pl.ds(base + off, _CG)])
        pltpu.sync_copy(ones, cnt_sp.at[idxr], add=True)
        return carry

    lax.fori_loop(0, _NCH, chunk, 0)
    for d in range(4):
        pltpu.sync_copy(planes.at[d], cdt.at[d, pl.ds(base, _EPW)])
    plsc.subcore_barrier()
    pltpu.sync_copy(cnt_sp.at[pl.ds(s * _TSL, _TSL)],
                    cntp.at[c, pl.ds(s * _TSL, _TSL)])


def _k2(pr, pc, cx, cy, cz, row, col, z1):
    mesh = plsc.VectorSubcoreMesh(core_axis_name="c", subcore_axis_name="s")
    f = pl.kernel(
        _k2_body,
        out_type=[
            jax.ShapeDtypeStruct((E, H), jnp.float32),
            jax.ShapeDtypeStruct((E, H), jnp.float32),
            jax.ShapeDtypeStruct((4, E), jnp.float32),
            jax.ShapeDtypeStruct((2, NPAD), jnp.float32),
        ],
        mesh=mesh,
        scratch_types=[
            pltpu.VMEM((_CG,), jnp.int32),
            pltpu.VMEM((_CG,), jnp.int32),
            pltpu.VMEM((_CG, H), jnp.float32),
            pltpu.VMEM((_CG, H), jnp.float32),
            pltpu.VMEM((_CG,), jnp.float32),
            pltpu.VMEM((_CG,), jnp.float32),
            pltpu.VMEM((_CG,), jnp.float32),
            pltpu.VMEM((_CG,), jnp.float32),
            pltpu.VMEM((_CG,), jnp.float32),
            pltpu.VMEM((_CG,), jnp.float32),
            pltpu.VMEM((4, _EPW), jnp.float32),
            pltpu.VMEM((_CG,), jnp.float32),
            pltpu.VMEM_SHARED((NPAD,), jnp.float32),
            pltpu.SemaphoreType.DMA,
        ],
        compiler_params=pltpu.CompilerParams(use_tc_tiling_on_sc=False),
    )
    return f(pr, pc, cx, cy, cz, row, col, z1)


NPAD = 10240
_TSL = NPAD // _NS
NAGG = 10000
_TSA = NAGG // _NS
_CS = 80
_SPLITS = ((0, 80),)


def _k4_body(ef, trt, row, z128, z1, aggp, trp,
             idxb, efb, trpl, agg_sp, t0_sp, t1_sp, t2_sp, sem):
    c = lax.axis_index("c")
    s = lax.axis_index("s")
    wid = s * _NC + c
    base = wid * _EPW
    tsp = [t0_sp, t1_sp, t2_sp]

    pltpu.sync_copy(z128, agg_sp.at[pl.ds(s * _TSA, _TSA)])
    for d in range(3):
        pltpu.sync_copy(z1, tsp[d].at[pl.ds(s * _TSL, _TSL)])
        pltpu.sync_copy(trt.at[d, pl.ds(base, _EPW)], trpl.at[d])
    plsc.subcore_barrier()

    def chunk(i, carry):
        off = base + i * _CS
        pltpu.sync_copy(row.at[pl.ds(off, _CS)], idxb)
        pltpu.sync_copy(ef.at[pl.ds(off, _CS)], efb)
        cps = []
        for (po, pn) in _SPLITS:
            sl = pl.ds(po, pn)
            ix = idxb.at[sl]
            cps.append(pltpu.async_copy(efb.at[sl], agg_sp.at[ix], sem,
                                        add=True))
            for d in range(3):
                cps.append(pltpu.async_copy(
                    trpl.at[d, pl.ds(i * _CS + po, pn)],
                    tsp[d].at[ix], sem, add=True))
        for cp in cps:
            cp.wait()
        return carry

    lax.fori_loop(0, _EPW // _CS, chunk, 0)
    plsc.subcore_barrier()

    pltpu.sync_copy(agg_sp.at[pl.ds(s * _TSA, _TSA)],
                    aggp.at[c, pl.ds(s * _TSA, _TSA)])
    for d in range(3):
        pltpu.sync_copy(tsp[d].at[pl.ds(s * _TSL, _TSL)],
                        trp.at[c, d, pl.ds(s * _TSL, _TSL)])


def _k4(ef, trt, row, z128, z1):
    mesh = plsc.VectorSubcoreMesh(core_axis_name="c", subcore_axis_name="s")
    f = pl.kernel(
        _k4_body,
        out_type=[
            jax.ShapeDtypeStruct((2, NAGG, H), jnp.float32),
            jax.ShapeDtypeStruct((2, 3, NPAD), jnp.float32),
        ],
        mesh=mesh,
        scratch_types=[
            pltpu.VMEM((_CS,), jnp.int32),
            pltpu.VMEM((_CS, H), jnp.float32),
            pltpu.VMEM((3, _EPW), jnp.float32),
            pltpu.VMEM_SHARED((NAGG, H), jnp.float32),
            pltpu.VMEM_SHARED((NPAD,), jnp.float32),
            pltpu.VMEM_SHARED((NPAD,), jnp.float32),
            pltpu.VMEM_SHARED((NPAD,), jnp.float32),
            pltpu.SemaphoreType.DMA,
        ],
        compiler_params=pltpu.CompilerParams(use_tc_tiling_on_sc=False),
    )
    return f(ef, trt, row, z128, z1)


def _k3_body(gr_ref, gc_ref, cd_ref, ea_ref,
             w8_ref, w1e_ref, b1_ref, w2_ref, b2_ref,
             lng_ref, lnb_ref, cw1_ref, cb1_ref, cw2_ref,
             ef_ref, trt_ref):
    cd = cd_ref[:]
    pre1 = (gr_ref[:] + gc_ref[:]
            + jax.lax.dot_general(cd, w8_ref[:], (((0,), (0,)), ((), ())),
                                  preferred_element_type=jnp.float32)
            + jax.lax.dot_general(ea_ref[:], w1e_ref[:], (((0,), (0,)), ((), ())),
                                  preferred_element_type=jnp.float32)
            + b1_ref[:])
    x1 = _silu(pre1)
    x2 = _silu(jnp.dot(x1, w2_ref[:], preferred_element_type=jnp.float32) + b2_ref[:])
    ef = _ln(x2, lng_ref[:], lnb_ref[:])
    ef_ref[:] = ef
    s2 = _silu(jnp.dot(ef, cw1_ref[:], preferred_element_type=jnp.float32) + cb1_ref[:])
    cm_row = jax.lax.dot_general(cw2_ref[:], s2, (((0,), (1,)), ((), ())),
                                 preferred_element_type=jnp.float32)
    trt_ref[:] = cd * cm_row


def _k3(gr, gc, cdt, ea_t, w8, w1e, b1, w2, b2, lng, lnb, cw1, cb1, cw2):
    grid = E // BE
    full = lambda shape: pl.BlockSpec(shape, lambda i: tuple(0 for _ in shape))
    return pl.pallas_call(
        _k3_body,
        grid=(grid,),
        in_specs=[
            pl.BlockSpec((BE, H), lambda i: (i, 0)),
            pl.BlockSpec((BE, H), lambda i: (i, 0)),
            pl.BlockSpec((4, BE), lambda i: (0, i)),
            pl.BlockSpec((DE, BE), lambda i: (0, i)),
            full((4, H)),
            full((DE, H)),
            full((H,)),
            full((H, H)),
            full((H,)),
            full((H,)),
            full((H,)),
            full((H, H)),
            full((H,)),
            full((H, 1)),
        ],
        out_specs=[
            pl.BlockSpec((BE, H), lambda i: (i, 0)),
            pl.BlockSpec((4, BE), lambda i: (0, i)),
        ],
        out_shape=[
            jax.ShapeDtypeStruct((E, H), jnp.float32),
            jax.ShapeDtypeStruct((4, E), jnp.float32),
        ],
    )(gr, gc, cdt, ea_t, w8, w1e, b1, w2, b2, lng, lnb, cw1, cb1, cw2)


def _k5_body(h_ref, hn_ref, agg_ref,
             w1h_ref, w1a_ref, b1_ref, w2_ref, b2_ref,
             hout_ref):
    agg = agg_ref[0] + agg_ref[1]
    pre = (jnp.dot(hn_ref[:], w1h_ref[:], preferred_element_type=jnp.float32)
           + jnp.dot(agg, w1a_ref[:], preferred_element_type=jnp.float32)
           + b1_ref[:])
    nh = jnp.dot(_silu(pre), w2_ref[:], preferred_element_type=jnp.float32) + b2_ref[:]
    hout_ref[:] = h_ref[:] + nh


def _k5(h, hn, aggp, w1h, w1a, b1, w2, b2):
    grid = N // BN
    full = lambda shape: pl.BlockSpec(shape, lambda i: tuple(0 for _ in shape))
    return pl.pallas_call(
        _k5_body,
        grid=(grid,),
        in_specs=[
            pl.BlockSpec((BN, D), lambda i: (i, 0)),
            pl.BlockSpec((BN, D), lambda i: (i, 0)),
            pl.BlockSpec((2, BN, H), lambda i: (0, i, 0)),
            full((D, H)),
            full((H, H)),
            full((H,)),
            full((H, D)),
            full((D,)),
        ],
        out_specs=[
            pl.BlockSpec((BN, D), lambda i: (i, 0)),
        ],
        out_shape=[
            jax.ShapeDtypeStruct((N, D), jnp.float32),
        ],
    )(h, hn, aggp, w1h, w1a, b1, w2, b2)


def _k6_body(trp_ref, cnt_ref, ct_ref, cout_ref):
    tr = trp_ref[0] + trp_ref[1]
    cnt = jnp.maximum(cnt_ref[0:1, :N] + cnt_ref[1:2, :N], 1.0)
    cout_ref[:] = ct_ref[:] + tr[0:3, :N] / cnt


def _k6(trp, cntp, coord_t):
    return pl.pallas_call(
        _k6_body,
        out_shape=jax.ShapeDtypeStruct((3, N), jnp.float32),
    )(trp, cntp, coord_t)


def kernel(h, edge_index, coord, edge_attr, node_ln_g, node_ln_b,
           edge_ln_g, edge_ln_b, e_W1, e_b1, e_W2, e_b2,
           n_W1, n_b1, n_W2, n_b2, c_W1, c_b1, c_W2):
    row = edge_index[0]
    col = edge_index[1]
    w1r = e_W1[0:D]
    w1c = e_W1[D:2 * D]
    wrad = e_W1[2 * D]
    w1e = e_W1[2 * D + 1:]
    w8 = jnp.zeros((4, H), jnp.float32).at[3].set(wrad)
    ea_t = edge_attr.T
    coord_t = coord.T

    hn, pr, pc = _k1(h, node_ln_g, node_ln_b, w1r, w1c)

    z1 = jnp.zeros((_TSL,), jnp.float32)
    gr, gc, cdt, cntp = _k2(pr, pc, coord_t[0], coord_t[1], coord_t[2],
                            row, col, z1)

    ef, trt = _k3(gr, gc, cdt, ea_t, w8, w1e, e_b1, e_W2, e_b2,
                  edge_ln_g, edge_ln_b, c_W1, c_b1, c_W2)

    z128 = jnp.zeros((_TSA, H), jnp.float32)
    aggp, trp = _k4(ef, trt, row, z128, z1)

    (h_out,) = _k5(h, hn, aggp, n_W1[0:D], n_W1[D:], n_b1, n_W2, n_b2)
    coord_out_t = _k6(trp, cntp, coord_t)
    return (h_out, coord_out_t.T, edge_attr)


# speedup vs baseline: 1.8166x; 1.1859x over previous
import functools

import jax
import jax.numpy as jnp
from jax import lax
from jax.experimental import pallas as pl
from jax.experimental.pallas import tpu as pltpu
from jax.experimental.pallas import tpu_sc as plsc

N = 10000
E = 320000
D = 128
H = 128
DE = 16

BN = 2000
BE = 2560

_NC = 2
_NS = 16
_NW = _NC * _NS
_CG = 80
_EPW = E // _NW
_NCH = _EPW // _CG
_L = 16


def _silu(x):
    return x / (1.0 + jnp.exp(-x))


def _ln(x, g, b, eps=1e-5):
    mu = jnp.mean(x, axis=-1, keepdims=True)
    var = jnp.mean((x - mu) ** 2, axis=-1, keepdims=True)
    return (x - mu) * jax.lax.rsqrt(var + eps) * g + b


def _k1_body(h_ref, g_ref, b_ref, w1r_ref, w1c_ref,
             hn_ref, pr_ref, pc_ref):
    h = h_ref[:]
    hn = _ln(h, g_ref[:], b_ref[:])
    hn_ref[:] = hn
    pr_ref[:] = jnp.dot(hn, w1r_ref[:], preferred_element_type=jnp.float32)
    pc_ref[:] = jnp.dot(hn, w1c_ref[:], preferred_element_type=jnp.float32)


def _k1(h, g, b, w1r, w1c):
    grid = N // BN
    return pl.pallas_call(
        _k1_body,
        grid=(grid,),
        in_specs=[
            pl.BlockSpec((BN, D), lambda i: (i, 0)),
            pl.BlockSpec((D,), lambda i: (0,)),
            pl.BlockSpec((D,), lambda i: (0,)),
            pl.BlockSpec((D, H), lambda i: (0, 0)),
            pl.BlockSpec((D, H), lambda i: (0, 0)),
        ],
        out_specs=[
            pl.BlockSpec((BN, D), lambda i: (i, 0)),
            pl.BlockSpec((BN, H), lambda i: (i, 0)),
            pl.BlockSpec((BN, H), lambda i: (i, 0)),
        ],
        out_shape=[
            jax.ShapeDtypeStruct((N, D), jnp.float32),
            jax.ShapeDtypeStruct((N, H), jnp.float32),
            jax.ShapeDtypeStruct((N, H), jnp.float32),
        ],
    )(h, g, b, w1r, w1c)


def _k2_body(eh, pr, pc, cx, cy, cz, row, col, z1, gr, gc, cdt, cntp,
             idxr, idxc, grb, gcb, crx, cry, crz, ccx, ccy, ccz,
             planes, ones, cnt_sp, sem):
    epw = eh // _NW
    c = lax.axis_index("c")
    s = lax.axis_index("s")
    wid = s * _NC + c
    base = wid * epw
    pltpu.sync_copy(z1, cnt_sp.at[pl.ds(s * _TSL, _TSL)])
    for j in range(_CG // _L):
        ones[pl.ds(j * _L, _L)] = jnp.full((_L,), 1.0, jnp.float32)
    plsc.subcore_barrier()

    def chunk(i, carry):
        off = i * _CG
        pltpu.sync_copy(row.at[pl.ds(base + off, _CG)], idxr)
        pltpu.sync_copy(col.at[pl.ds(base + off, _CG)], idxc)
        cps = [
            pltpu.async_copy(pr.at[idxr], grb, sem),
            pltpu.async_copy(pc.at[idxc], gcb, sem),
            pltpu.async_copy(cx.at[idxr], crx, sem),
            pltpu.async_copy(cy.at[idxr], cry, sem),
            pltpu.async_copy(cz.at[idxr], crz, sem),
            pltpu.async_copy(cx.at[idxc], ccx, sem),
            pltpu.async_copy(cy.at[idxc], ccy, sem),
            pltpu.async_copy(cz.at[idxc], ccz, sem),
        ]
        for cp in cps:
            cp.wait()
        for j in range(_CG // _L):
            sj = pl.ds(j * _L, _L)
            so = pl.ds(off + j * _L, _L)
            dx = crx[sj] - ccx[sj]
            dy = cry[sj] - ccy[sj]
            dz = crz[sj] - ccz[sj]
            planes[0, so] = dx
            planes[1, so] = dy
            planes[2, so] = dz
            planes[3, so] = dx * dx + dy * dy + dz * dz
        pltpu.sync_copy(grb, gr.at[pl.ds(base + off, _CG)])
        pltpu.sync_copy(gcb, gc.at[pl.ds(base + off, _CG)])
        pltpu.sync_copy(ones, cnt_sp.at[idxr], add=True)
        return carry

    lax.fori_loop(0, epw // _CG, chunk, 0)
    for d in range(4):
        pltpu.sync_copy(planes.at[d], cdt.at[d, pl.ds(base, epw)])
    plsc.subcore_barrier()
    pltpu.sync_copy(cnt_sp.at[pl.ds(s * _TSL, _TSL)],
                    cntp.at[c, pl.ds(s * _TSL, _TSL)])


def _k2(pr, pc, cx, cy, cz, row, col, z1, eh):
    mesh = plsc.VectorSubcoreMesh(core_axis_name="c", subcore_axis_name="s")
    f = pl.kernel(
        functools.partial(_k2_body, eh),
        out_type=[
            jax.ShapeDtypeStruct((eh, H), jnp.float32),
            jax.ShapeDtypeStruct((eh, H), jnp.float32),
            jax.ShapeDtypeStruct((4, eh), jnp.float32),
            jax.ShapeDtypeStruct((2, NPAD), jnp.float32),
        ],
        mesh=mesh,
        scratch_types=[
            pltpu.VMEM((_CG,), jnp.int32),
            pltpu.VMEM((_CG,), jnp.int32),
            pltpu.VMEM((_CG, H), jnp.float32),
            pltpu.VMEM((_CG, H), jnp.float32),
            pltpu.VMEM((_CG,), jnp.float32),
            pltpu.VMEM((_CG,), jnp.float32),
            pltpu.VMEM((_CG,), jnp.float32),
            pltpu.VMEM((_CG,), jnp.float32),
            pltpu.VMEM((_CG,), jnp.float32),
            pltpu.VMEM((_CG,), jnp.float32),
            pltpu.VMEM((4, eh // _NW), jnp.float32),
            pltpu.VMEM((_CG,), jnp.float32),
            pltpu.VMEM_SHARED((NPAD,), jnp.float32),
            pltpu.SemaphoreType.DMA,
        ],
        compiler_params=pltpu.CompilerParams(use_tc_tiling_on_sc=False),
    )
    return f(pr, pc, cx, cy, cz, row, col, z1)


NPAD = 10240
_TSL = NPAD // _NS
NAGG = 10000
_TSA = NAGG // _NS
_CS = 80
_SPLITS = ((0, 80),)


def _k4_body(eh, ef, trt, row, z128, z1, aggp, trp,
             idxb, efb, trpl, agg_sp, t0_sp, t1_sp, t2_sp, sem):
    epw = eh // _NW
    c = lax.axis_index("c")
    s = lax.axis_index("s")
    wid = s * _NC + c
    base = wid * epw
    tsp = [t0_sp, t1_sp, t2_sp]

    pltpu.sync_copy(z128, agg_sp.at[pl.ds(s * _TSA, _TSA)])
    for d in range(3):
        pltpu.sync_copy(z1, tsp[d].at[pl.ds(s * _TSL, _TSL)])
        pltpu.sync_copy(trt.at[d, pl.ds(base, epw)], trpl.at[d])
    plsc.subcore_barrier()

    def chunk(i, carry):
        off = base + i * _CS
        pltpu.sync_copy(row.at[pl.ds(off, _CS)], idxb)
        pltpu.sync_copy(ef.at[pl.ds(off, _CS)], efb)
        cps = []
        for (po, pn) in _SPLITS:
            sl = pl.ds(po, pn)
            ix = idxb.at[sl]
            cps.append(pltpu.async_copy(efb.at[sl], agg_sp.at[ix], sem,
                                        add=True))
            for d in range(3):
                cps.append(pltpu.async_copy(
                    trpl.at[d, pl.ds(i * _CS + po, pn)],
                    tsp[d].at[ix], sem, add=True))
        for cp in cps:
            cp.wait()
        return carry

    lax.fori_loop(0, epw // _CS, chunk, 0)
    plsc.subcore_barrier()

    pltpu.sync_copy(agg_sp.at[pl.ds(s * _TSA, _TSA)],
                    aggp.at[c, pl.ds(s * _TSA, _TSA)])
    for d in range(3):
        pltpu.sync_copy(tsp[d].at[pl.ds(s * _TSL, _TSL)],
                        trp.at[c, d, pl.ds(s * _TSL, _TSL)])


def _k4(ef, trt, row, z128, z1, eh):
    mesh = plsc.VectorSubcoreMesh(core_axis_name="c", subcore_axis_name="s")
    f = pl.kernel(
        functools.partial(_k4_body, eh),
        out_type=[
            jax.ShapeDtypeStruct((2, NAGG, H), jnp.float32),
            jax.ShapeDtypeStruct((2, 3, NPAD), jnp.float32),
        ],
        mesh=mesh,
        scratch_types=[
            pltpu.VMEM((_CS,), jnp.int32),
            pltpu.VMEM((_CS, H), jnp.float32),
            pltpu.VMEM((3, eh // _NW), jnp.float32),
            pltpu.VMEM_SHARED((NAGG, H), jnp.float32),
            pltpu.VMEM_SHARED((NPAD,), jnp.float32),
            pltpu.VMEM_SHARED((NPAD,), jnp.float32),
            pltpu.VMEM_SHARED((NPAD,), jnp.float32),
            pltpu.SemaphoreType.DMA,
        ],
        compiler_params=pltpu.CompilerParams(use_tc_tiling_on_sc=False),
    )
    return f(ef, trt, row, z128, z1)


def _k3_body(gr_ref, gc_ref, cd_ref, ea_ref,
             w8_ref, w1e_ref, b1_ref, w2_ref, b2_ref,
             lng_ref, lnb_ref, cw1_ref, cb1_ref, cw2_ref,
             ef_ref, trt_ref):
    cd = cd_ref[:]
    pre1 = (gr_ref[:] + gc_ref[:]
            + jax.lax.dot_general(cd, w8_ref[:], (((0,), (0,)), ((), ())),
                                  preferred_element_type=jnp.float32)
            + jax.lax.dot_general(ea_ref[:], w1e_ref[:], (((0,), (0,)), ((), ())),
                                  preferred_element_type=jnp.float32)
            + b1_ref[:])
    x1 = _silu(pre1)
    x2 = _silu(jnp.dot(x1, w2_ref[:], preferred_element_type=jnp.float32) + b2_ref[:])
    ef = _ln(x2, lng_ref[:], lnb_ref[:])
    ef_ref[:] = ef
    s2 = _silu(jnp.dot(ef, cw1_ref[:], preferred_element_type=jnp.float32) + cb1_ref[:])
    cm_row = jax.lax.dot_general(cw2_ref[:], s2, (((0,), (1,)), ((), ())),
                                 preferred_element_type=jnp.float32)
    trt_ref[:] = cd * cm_row


def _k3(gr, gc, cdt, ea_t, w8, w1e, b1, w2, b2, lng, lnb, cw1, cb1, cw2):
    eh = gr.shape[0]
    grid = eh // BE
    full = lambda shape: pl.BlockSpec(shape, lambda i: tuple(0 for _ in shape))
    return pl.pallas_call(
        _k3_body,
        grid=(grid,),
        in_specs=[
            pl.BlockSpec((BE, H), lambda i: (i, 0)),
            pl.BlockSpec((BE, H), lambda i: (i, 0)),
            pl.BlockSpec((4, BE), lambda i: (0, i)),
            pl.BlockSpec((DE, BE), lambda i: (0, i)),
            full((4, H)),
            full((DE, H)),
            full((H,)),
            full((H, H)),
            full((H,)),
            full((H,)),
            full((H,)),
            full((H, H)),
            full((H,)),
            full((H, 1)),
        ],
        out_specs=[
            pl.BlockSpec((BE, H), lambda i: (i, 0)),
            pl.BlockSpec((4, BE), lambda i: (0, i)),
        ],
        out_shape=[
            jax.ShapeDtypeStruct((eh, H), jnp.float32),
            jax.ShapeDtypeStruct((4, eh), jnp.float32),
        ],
    )(gr, gc, cdt, ea_t, w8, w1e, b1, w2, b2, lng, lnb, cw1, cb1, cw2)


def _k5_body(h_ref, hn_ref, agg_ref, aggb_ref,
             w1h_ref, w1a_ref, b1_ref, w2_ref, b2_ref,
             hout_ref):
    agg = agg_ref[0] + agg_ref[1] + aggb_ref[0] + aggb_ref[1]
    pre = (jnp.dot(hn_ref[:], w1h_ref[:], preferred_element_type=jnp.float32)
           + jnp.dot(agg, w1a_ref[:], preferred_element_type=jnp.float32)
           + b1_ref[:])
    nh = jnp.dot(_silu(pre), w2_ref[:], preferred_element_type=jnp.float32) + b2_ref[:]
    hout_ref[:] = h_ref[:] + nh


def _k5(h, hn, aggp, aggpb, w1h, w1a, b1, w2, b2):
    grid = N // BN
    full = lambda shape: pl.BlockSpec(shape, lambda i: tuple(0 for _ in shape))
    return pl.pallas_call(
        _k5_body,
        grid=(grid,),
        in_specs=[
            pl.BlockSpec((BN, D), lambda i: (i, 0)),
            pl.BlockSpec((BN, D), lambda i: (i, 0)),
            pl.BlockSpec((2, BN, H), lambda i: (0, i, 0)),
            pl.BlockSpec((2, BN, H), lambda i: (0, i, 0)),
            full((D, H)),
            full((H, H)),
            full((H,)),
            full((H, D)),
            full((D,)),
        ],
        out_specs=[
            pl.BlockSpec((BN, D), lambda i: (i, 0)),
        ],
        out_shape=[
            jax.ShapeDtypeStruct((N, D), jnp.float32),
        ],
    )(h, hn, aggp, aggpb, w1h, w1a, b1, w2, b2)


def _k6_body(trp_ref, trpb_ref, cnt_ref, cntb_ref, ct_ref, cout_ref):
    tr = trp_ref[0] + trp_ref[1] + trpb_ref[0] + trpb_ref[1]
    cnt = jnp.maximum(cnt_ref[0:1, :N] + cnt_ref[1:2, :N]
                      + cntb_ref[0:1, :N] + cntb_ref[1:2, :N], 1.0)
    cout_ref[:] = ct_ref[:] + tr[0:3, :N] / cnt


def _k6(trp, trpb, cntp, cntpb, coord_t):
    return pl.pallas_call(
        _k6_body,
        out_shape=jax.ShapeDtypeStruct((3, N), jnp.float32),
    )(trp, trpb, cntp, cntpb, coord_t)


def kernel(h, edge_index, coord, edge_attr, node_ln_g, node_ln_b,
           edge_ln_g, edge_ln_b, e_W1, e_b1, e_W2, e_b2,
           n_W1, n_b1, n_W2, n_b2, c_W1, c_b1, c_W2):
    row = edge_index[0]
    col = edge_index[1]
    w1r = e_W1[0:D]
    w1c = e_W1[D:2 * D]
    wrad = e_W1[2 * D]
    w1e = e_W1[2 * D + 1:]
    w8 = jnp.zeros((4, H), jnp.float32).at[3].set(wrad)
    ea_t = edge_attr.T
    coord_t = coord.T

    hn, pr, pc = _k1(h, node_ln_g, node_ln_b, w1r, w1c)

    z1 = jnp.zeros((_TSL,), jnp.float32)
    z128 = jnp.zeros((_TSA, H), jnp.float32)
    EH0 = 163840
    halves = []
    for (lo, eh) in ((0, EH0), (EH0, E - EH0)):
        rw = lax.slice_in_dim(row, lo, lo + eh)
        cl = lax.slice_in_dim(col, lo, lo + eh)
        gr, gc, cdt, cntp = _k2(pr, pc, coord_t[0], coord_t[1], coord_t[2],
                                rw, cl, z1, eh)
        ef, trt = _k3(gr, gc, cdt, ea_t[:, lo:lo + eh], w8, w1e, e_b1,
                      e_W2, e_b2, edge_ln_g, edge_ln_b, c_W1, c_b1, c_W2)
        aggp, trp = _k4(ef, trt, rw, z128, z1, eh)
        halves.append((aggp, trp, cntp))

    (h_out,) = _k5(h, hn, halves[0][0], halves[1][0],
                   n_W1[0:D], n_W1[D:], n_b1, n_W2, n_b2)
    coord_out_t = _k6(halves[0][1], halves[1][1], halves[0][2], halves[1][2],
                      coord_t)
    return (h_out, coord_out_t.T, edge_attr)


# speedup vs baseline: 1.9595x; 1.0787x over previous
import functools

import jax
import jax.numpy as jnp
from jax import lax
from jax.experimental import pallas as pl
from jax.experimental.pallas import tpu as pltpu
from jax.experimental.pallas import tpu_sc as plsc

N = 10000
E = 320000
D = 128
H = 128
DE = 16

BN = 2000
BE = 2560

_NC = 2
_NS = 16
_NW = _NC * _NS
_CG = 80
_EPW = E // _NW
_NCH = _EPW // _CG
_L = 16


def _silu(x):
    return x / (1.0 + jnp.exp(-x))


def _ln(x, g, b, eps=1e-5):
    mu = jnp.mean(x, axis=-1, keepdims=True)
    var = jnp.mean((x - mu) ** 2, axis=-1, keepdims=True)
    return (x - mu) * jax.lax.rsqrt(var + eps) * g + b


def _k1_body(h_ref, g_ref, b_ref, w1r_ref, w1c_ref,
             hn_ref, pr_ref, pc_ref):
    h = h_ref[:]
    hn = _ln(h, g_ref[:], b_ref[:])
    hn_ref[:] = hn
    pr_ref[:] = jnp.dot(hn, w1r_ref[:], preferred_element_type=jnp.float32)
    pc_ref[:] = jnp.dot(hn, w1c_ref[:], preferred_element_type=jnp.float32)


def _k1(h, g, b, w1r, w1c):
    grid = N // BN
    return pl.pallas_call(
        _k1_body,
        grid=(grid,),
        in_specs=[
            pl.BlockSpec((BN, D), lambda i: (i, 0)),
            pl.BlockSpec((D,), lambda i: (0,)),
            pl.BlockSpec((D,), lambda i: (0,)),
            pl.BlockSpec((D, H), lambda i: (0, 0)),
            pl.BlockSpec((D, H), lambda i: (0, 0)),
        ],
        out_specs=[
            pl.BlockSpec((BN, D), lambda i: (i, 0)),
            pl.BlockSpec((BN, H), lambda i: (i, 0)),
            pl.BlockSpec((BN, H), lambda i: (i, 0)),
        ],
        out_shape=[
            jax.ShapeDtypeStruct((N, D), jnp.float32),
            jax.ShapeDtypeStruct((N, H), jnp.float32),
            jax.ShapeDtypeStruct((N, H), jnp.float32),
        ],
    )(h, g, b, w1r, w1c)


def _k2_body(eh, pr, pc, cx, cy, cz, row, col, z1, gr, gc, cdt, cntp,
             idxr, idxc, grb, gcb, crx, cry, crz, ccx, ccy, ccz,
             planes, ones, cnt_sp, sem):
    epw = eh // _NW
    c = lax.axis_index("c")
    s = lax.axis_index("s")
    wid = s * _NC + c
    base = wid * epw
    pltpu.sync_copy(z1, cnt_sp.at[pl.ds(s * _TSL, _TSL)])
    for j in range(_CG // _L):
        ones[pl.ds(j * _L, _L)] = jnp.full((_L,), 1.0, jnp.float32)
    plsc.subcore_barrier()

    def chunk(i, carry):
        off = i * _CG
        pltpu.sync_copy(row.at[pl.ds(base + off, _CG)], idxr)
        pltpu.sync_copy(col.at[pl.ds(base + off, _CG)], idxc)
        cps = [
            pltpu.async_copy(pr.at[idxr], grb, sem),
            pltpu.async_copy(pc.at[idxc], gcb, sem),
            pltpu.async_copy(cx.at[idxr], crx, sem),
            pltpu.async_copy(cy.at[idxr], cry, sem),
            pltpu.async_copy(cz.at[idxr], crz, sem),
            pltpu.async_copy(cx.at[idxc], ccx, sem),
            pltpu.async_copy(cy.at[idxc], ccy, sem),
            pltpu.async_copy(cz.at[idxc], ccz, sem),
        ]
        for cp in cps:
            cp.wait()
        for j in range(_CG // _L):
            sj = pl.ds(j * _L, _L)
            so = pl.ds(off + j * _L, _L)
            dx = crx[sj] - ccx[sj]
            dy = cry[sj] - ccy[sj]
            dz = crz[sj] - ccz[sj]
            planes[0, so] = dx
            planes[1, so] = dy
            planes[2, so] = dz
            planes[3, so] = dx * dx + dy * dy + dz * dz
        pltpu.sync_copy(grb, gr.at[pl.ds(base + off, _CG)])
        pltpu.sync_copy(gcb, gc.at[pl.ds(base + off, _CG)])
        pltpu.sync_copy(ones, cnt_sp.at[idxr], add=True)
        return carry

    lax.fori_loop(0, epw // _CG, chunk, 0)
    for d in range(4):
        pltpu.sync_copy(planes.at[d], cdt.at[d, pl.ds(base, epw)])
    plsc.subcore_barrier()
    pltpu.sync_copy(cnt_sp.at[pl.ds(s * _TSL, _TSL)],
                    cntp.at[c, pl.ds(s * _TSL, _TSL)])


def _k2(pr, pc, cx, cy, cz, row, col, z1, eh):
    mesh = plsc.VectorSubcoreMesh(core_axis_name="c", subcore_axis_name="s")
    f = pl.kernel(
        functools.partial(_k2_body, eh),
        out_type=[
            jax.ShapeDtypeStruct((eh, H), jnp.float32),
            jax.ShapeDtypeStruct((eh, H), jnp.float32),
            jax.ShapeDtypeStruct((4, eh), jnp.float32),
            jax.ShapeDtypeStruct((2, NPAD), jnp.float32),
        ],
        mesh=mesh,
        scratch_types=[
            pltpu.VMEM((_CG,), jnp.int32),
            pltpu.VMEM((_CG,), jnp.int32),
            pltpu.VMEM((_CG, H), jnp.float32),
            pltpu.VMEM((_CG, H), jnp.float32),
            pltpu.VMEM((_CG,), jnp.float32),
            pltpu.VMEM((_CG,), jnp.float32),
            pltpu.VMEM((_CG,), jnp.float32),
            pltpu.VMEM((_CG,), jnp.float32),
            pltpu.VMEM((_CG,), jnp.float32),
            pltpu.VMEM((_CG,), jnp.float32),
            pltpu.VMEM((4, eh // _NW), jnp.float32),
            pltpu.VMEM((_CG,), jnp.float32),
            pltpu.VMEM_SHARED((NPAD,), jnp.float32),
            pltpu.SemaphoreType.DMA,
        ],
        compiler_params=pltpu.CompilerParams(use_tc_tiling_on_sc=False),
    )
    return f(pr, pc, cx, cy, cz, row, col, z1)


NPAD = 10240
_TSL = NPAD // _NS
NAGG = 10000
_TSA = NAGG // _NS
_CS = 80
_SPLITS = ((0, 80),)


def _k4_body(eh, ef, trt, row, z128, z1, aggp, trp,
             idxb, efb, idxb2, efb2, trpl, agg_sp, t0_sp, t1_sp, t2_sp,
             sem, sem2, sem3):
    epw = eh // _NW
    c = lax.axis_index("c")
    s = lax.axis_index("s")
    wid = s * _NC + c
    base = wid * epw
    tsp = [t0_sp, t1_sp, t2_sp]

    pltpu.sync_copy(z128, agg_sp.at[pl.ds(s * _TSA, _TSA)])
    for d in range(3):
        pltpu.sync_copy(z1, tsp[d].at[pl.ds(s * _TSL, _TSL)])
        pltpu.sync_copy(trt.at[d, pl.ds(base, epw)], trpl.at[d])
    plsc.subcore_barrier()

    def pair(i, carry):
        offa = base + (2 * i) * _CS
        offb = offa + _CS
        la = [pltpu.async_copy(row.at[pl.ds(offa, _CS)], idxb, sem2),
              pltpu.async_copy(ef.at[pl.ds(offa, _CS)], efb, sem2)]
        lb = [pltpu.async_copy(row.at[pl.ds(offb, _CS)], idxb2, sem3),
              pltpu.async_copy(ef.at[pl.ds(offb, _CS)], efb2, sem3)]
        cps = []

        def scat(j, ib, eb):
            for (po, pn) in _SPLITS:
                sl = pl.ds(po, pn)
                ix = ib.at[sl]
                cps.append(pltpu.async_copy(eb.at[sl], agg_sp.at[ix], sem,
                                            add=True))
                for d in range(3):
                    cps.append(pltpu.async_copy(
                        trpl.at[d, pl.ds(j * _CS + po, pn)],
                        tsp[d].at[ix], sem, add=True))

        for cp in la:
            cp.wait()
        scat(2 * i, idxb, efb)
        for cp in lb:
            cp.wait()
        scat(2 * i + 1, idxb2, efb2)
        for cp in cps:
            cp.wait()
        return carry

    lax.fori_loop(0, epw // _CS // 2, pair, 0)
    plsc.subcore_barrier()

    pltpu.sync_copy(agg_sp.at[pl.ds(s * _TSA, _TSA)],
                    aggp.at[c, pl.ds(s * _TSA, _TSA)])
    for d in range(3):
        pltpu.sync_copy(tsp[d].at[pl.ds(s * _TSL, _TSL)],
                        trp.at[c, d, pl.ds(s * _TSL, _TSL)])


def _k4(ef, trt, row, z128, z1, eh):
    mesh = plsc.VectorSubcoreMesh(core_axis_name="c", subcore_axis_name="s")
    f = pl.kernel(
        functools.partial(_k4_body, eh),
        out_type=[
            jax.ShapeDtypeStruct((2, NAGG, H), jnp.float32),
            jax.ShapeDtypeStruct((2, 3, NPAD), jnp.float32),
        ],
        mesh=mesh,
        scratch_types=[
            pltpu.VMEM((_CS,), jnp.int32),
            pltpu.VMEM((_CS, H), jnp.float32),
            pltpu.VMEM((_CS,), jnp.int32),
            pltpu.VMEM((_CS, H), jnp.float32),
            pltpu.VMEM((3, eh // _NW), jnp.float32),
            pltpu.VMEM_SHARED((NAGG, H), jnp.float32),
            pltpu.VMEM_SHARED((NPAD,), jnp.float32),
            pltpu.VMEM_SHARED((NPAD,), jnp.float32),
            pltpu.VMEM_SHARED((NPAD,), jnp.float32),
            pltpu.SemaphoreType.DMA,
            pltpu.SemaphoreType.DMA,
            pltpu.SemaphoreType.DMA,
        ],
        compiler_params=pltpu.CompilerParams(use_tc_tiling_on_sc=False),
    )
    return f(ef, trt, row, z128, z1)


def _k3_body(gr_ref, gc_ref, cd_ref, ea_ref,
             w8_ref, w1e_ref, b1_ref, w2_ref, b2_ref,
             lng_ref, lnb_ref, cw1_ref, cb1_ref, cw2_ref,
             ef_ref, trt_ref):
    cd = cd_ref[:]
    pre1 = (gr_ref[:] + gc_ref[:]
            + jax.lax.dot_general(cd, w8_ref[:], (((0,), (0,)), ((), ())),
                                  preferred_element_type=jnp.float32)
            + jax.lax.dot_general(ea_ref[:], w1e_ref[:], (((0,), (0,)), ((), ())),
                                  preferred_element_type=jnp.float32)
            + b1_ref[:])
    x1 = _silu(pre1)
    x2 = _silu(jnp.dot(x1, w2_ref[:], preferred_element_type=jnp.float32) + b2_ref[:])
    ef = _ln(x2, lng_ref[:], lnb_ref[:])
    ef_ref[:] = ef
    s2 = _silu(jnp.dot(ef, cw1_ref[:], preferred_element_type=jnp.float32) + cb1_ref[:])
    cm_row = jax.lax.dot_general(cw2_ref[:], s2, (((0,), (1,)), ((), ())),
                                 preferred_element_type=jnp.float32)
    trt_ref[:] = cd * cm_row


def _k3(gr, gc, cdt, ea_t, w8, w1e, b1, w2, b2, lng, lnb, cw1, cb1, cw2):
    eh = gr.shape[0]
    grid = eh // BE
    full = lambda shape: pl.BlockSpec(shape, lambda i: tuple(0 for _ in shape))
    return pl.pallas_call(
        _k3_body,
        grid=(grid,),
        in_specs=[
            pl.BlockSpec((BE, H), lambda i: (i, 0)),
            pl.BlockSpec((BE, H), lambda i: (i, 0)),
            pl.BlockSpec((4, BE), lambda i: (0, i)),
            pl.BlockSpec((DE, BE), lambda i: (0, i)),
            full((4, H)),
            full((DE, H)),
            full((H,)),
            full((H, H)),
            full((H,)),
            full((H,)),
            full((H,)),
            full((H, H)),
            full((H,)),
            full((H, 1)),
        ],
        out_specs=[
            pl.BlockSpec((BE, H), lambda i: (i, 0)),
            pl.BlockSpec((4, BE), lambda i: (0, i)),
        ],
        out_shape=[
            jax.ShapeDtypeStruct((eh, H), jnp.float32),
            jax.ShapeDtypeStruct((4, eh), jnp.float32),
        ],
    )(gr, gc, cdt, ea_t, w8, w1e, b1, w2, b2, lng, lnb, cw1, cb1, cw2)


def _k5_body(h_ref, hn_ref, agg_ref, aggb_ref,
             w1h_ref, w1a_ref, b1_ref, w2_ref, b2_ref,
             hout_ref):
    agg = agg_ref[0] + agg_ref[1] + aggb_ref[0] + aggb_ref[1]
    pre = (jnp.dot(hn_ref[:], w1h_ref[:], preferred_element_type=jnp.float32)
           + jnp.dot(agg, w1a_ref[:], preferred_element_type=jnp.float32)
           + b1_ref[:])
    nh = jnp.dot(_silu(pre), w2_ref[:], preferred_element_type=jnp.float32) + b2_ref[:]
    hout_ref[:] = h_ref[:] + nh


def _k5(h, hn, aggp, aggpb, w1h, w1a, b1, w2, b2):
    grid = N // BN
    full = lambda shape: pl.BlockSpec(shape, lambda i: tuple(0 for _ in shape))
    return pl.pallas_call(
        _k5_body,
        grid=(grid,),
        in_specs=[
            pl.BlockSpec((BN, D), lambda i: (i, 0)),
            pl.BlockSpec((BN, D), lambda i: (i, 0)),
            pl.BlockSpec((2, BN, H), lambda i: (0, i, 0)),
            pl.BlockSpec((2, BN, H), lambda i: (0, i, 0)),
            full((D, H)),
            full((H, H)),
            full((H,)),
            full((H, D)),
            full((D,)),
        ],
        out_specs=[
            pl.BlockSpec((BN, D), lambda i: (i, 0)),
        ],
        out_shape=[
            jax.ShapeDtypeStruct((N, D), jnp.float32),
        ],
    )(h, hn, aggp, aggpb, w1h, w1a, b1, w2, b2)


def _k6_body(trp_ref, trpb_ref, cnt_ref, cntb_ref, ct_ref, cout_ref):
    tr = trp_ref[0] + trp_ref[1] + trpb_ref[0] + trpb_ref[1]
    cnt = jnp.maximum(cnt_ref[0:1, :N] + cnt_ref[1:2, :N]
                      + cntb_ref[0:1, :N] + cntb_ref[1:2, :N], 1.0)
    cout_ref[:] = ct_ref[:] + tr[0:3, :N] / cnt


def _k6(trp, trpb, cntp, cntpb, coord_t):
    return pl.pallas_call(
        _k6_body,
        out_shape=jax.ShapeDtypeStruct((3, N), jnp.float32),
    )(trp, trpb, cntp, cntpb, coord_t)


def kernel(h, edge_index, coord, edge_attr, node_ln_g, node_ln_b,
           edge_ln_g, edge_ln_b, e_W1, e_b1, e_W2, e_b2,
           n_W1, n_b1, n_W2, n_b2, c_W1, c_b1, c_W2):
    row = edge_index[0]
    col = edge_index[1]
    w1r = e_W1[0:D]
    w1c = e_W1[D:2 * D]
    wrad = e_W1[2 * D]
    w1e = e_W1[2 * D + 1:]
    w8 = jnp.zeros((4, H), jnp.float32).at[3].set(wrad)
    ea_t = edge_attr.T
    coord_t = coord.T

    hn, pr, pc = _k1(h, node_ln_g, node_ln_b, w1r, w1c)

    z1 = jnp.zeros((_TSL,), jnp.float32)
    z128 = jnp.zeros((_TSA, H), jnp.float32)
    EH0 = 163840
    halves = []
    for (lo, eh) in ((0, EH0), (EH0, E - EH0)):
        rw = lax.slice_in_dim(row, lo, lo + eh)
        cl = lax.slice_in_dim(col, lo, lo + eh)
        gr, gc, cdt, cntp = _k2(pr, pc, coord_t[0], coord_t[1], coord_t[2],
                                rw, cl, z1, eh)
        ef, trt = _k3(gr, gc, cdt, ea_t[:, lo:lo + eh], w8, w1e, e_b1,
                      e_W2, e_b2, edge_ln_g, edge_ln_b, c_W1, c_b1, c_W2)
        aggp, trp = _k4(ef, trt, rw, z128, z1, eh)
        halves.append((aggp, trp, cntp))

    (h_out,) = _k5(h, hn, halves[0][0], halves[1][0],
                   n_W1[0:D], n_W1[D:], n_b1, n_W2, n_b2)
    coord_out_t = _k6(halves[0][1], halves[1][1], halves[0][2], halves[1][2],
                      coord_t)
    return (h_out, coord_out_t.T, edge_attr)
